# baseline (speedup 1.0000x reference)
import functools

import jax
import jax.numpy as jnp
from jax import lax
from jax.experimental import pallas as pl
from jax.experimental.pallas import tpu as pltpu
from jax.experimental.pallas import tpu_sc as plsc

N = 10000
E = 320000
DIN = 128
HID = 32
OUT = 3
EPS = 1e-5

NC = 2
NS = 16
NW = NC * NS
E_PER_TILE = E // NW
CH = 80
NCH = E_PER_TILE // CH
ZCH = 80
ROWS_A = 640
ROWS_B = N - (NS - 1) * ROWS_A
NCH_A = ROWS_A // ZCH
NCH_B = ROWS_B // ZCH


def _sc_aggregate(x, src, dst, z128, z1):
    mesh = plsc.VectorSubcoreMesh(core_axis_name="c", subcore_axis_name="s")

    @functools.partial(
        pl.kernel,
        out_type=[
            jax.ShapeDtypeStruct((NC, N, DIN), jnp.float32),
            jax.ShapeDtypeStruct((NC * N,), jnp.float32),
        ],
        mesh=mesh,
        scratch_types=[
            pltpu.VMEM((6, CH), jnp.int32),
            pltpu.VMEM((NCH, CH), jnp.int32),
            pltpu.VMEM((CH, DIN), jnp.float32),
            pltpu.VMEM((CH, DIN), jnp.float32),
            pltpu.VMEM((CH, DIN), jnp.float32),
            pltpu.VMEM((112,), jnp.float32),
            pltpu.VMEM((ROWS_A,), jnp.float32),
            pltpu.VMEM_SHARED((N, DIN), jnp.float32),
            pltpu.VMEM_SHARED((N,), jnp.float32),
            pltpu.SemaphoreType.DMA,
            pltpu.SemaphoreType.DMA,
            pltpu.SemaphoreType.DMA,
            pltpu.SemaphoreType.DMA,
            pltpu.SemaphoreType.DMA,
            pltpu.SemaphoreType.DMA,
            pltpu.SemaphoreType.DMA,
            pltpu.SemaphoreType.DMA,
            pltpu.SemaphoreType.DMA,
            pltpu.SemaphoreType.DMA,
            pltpu.SemaphoreType.DMA,
        ],
    )
    def agg(x_hbm, src_hbm, dst_hbm, z128_hbm, z1_hbm, sums_hbm, cnts_hbm,
            sidx, dstb, rows0, rows1, rows2, ones_v, cstg_v, acc_sh, cnt_sh,
            gs0, gs1, gs2, is0, is1, is2, is3, is4, is5, cs0, cs1):
        cid = lax.axis_index("c")
        sid = lax.axis_index("s")
        wid = cid * NS + sid
        r0 = sid * ROWS_A
        last = sid == NS - 1
        nch = jnp.where(last, NCH_B, NCH_A)

        gsems = [gs0, gs1, gs2]
        isems = [is0, is1, is2, is3, is4, is5]
        csems = [cs0, cs1]
        rows = [rows0, rows1, rows2]

        pltpu.sync_copy(dst_hbm.at[wid], dstb)

        pltpu.sync_copy(z128_hbm, rows0.at[pl.ds(0, ZCH)])
        pltpu.sync_copy(z1_hbm, cstg_v)

        @pl.loop(0, nch)
        def _(j):
            pltpu.sync_copy(rows0.at[pl.ds(0, ZCH)],
                            acc_sh.at[pl.ds(r0 + j * ZCH, ZCH)])

        @pl.when(last)
        def _():
            pltpu.sync_copy(cstg_v.at[pl.ds(0, ROWS_B)], cnt_sh.at[pl.ds(r0, ROWS_B)])

        @pl.when(jnp.logical_not(last))
        def _():
            pltpu.sync_copy(cstg_v, cnt_sh.at[pl.ds(r0, ROWS_A)])

        ovec = jnp.ones((16,), jnp.float32)

        @pl.loop(0, 7)
        def _(i):
            ones_v[pl.ds(i * 16, 16)] = ovec

        plsc.subcore_barrier()

        def ifire(j, b6):
            pltpu.async_copy(src_hbm.at[pl.ds(wid * E_PER_TILE + j * CH, CH)],
                             sidx.at[b6], isems[b6])

        def idrain(b6):
            pltpu.make_async_copy(src_hbm.at[pl.ds(0, CH)],
                                  sidx.at[b6], isems[b6]).wait()

        def fire(j, b3, b6):
            idrain(b6)
            pltpu.async_copy(x_hbm.at[sidx.at[b6]], rows[b3], gsems[b3])

        def drain(b3):
            pltpu.make_async_copy(x_hbm.at[sidx.at[0]], rows[b3],
                                  gsems[b3]).wait()

        def cfire(j, b2):
            pltpu.async_copy(ones_v.at[pl.ds(0, CH)], cnt_sh.at[dstb.at[j]],
                             csems[b2], add=True)

        def cdrain(b2):
            pltpu.make_async_copy(z1_hbm.at[pl.ds(0, CH)],
                                  cstg_v.at[pl.ds(0, CH)], csems[b2]).wait()

        for s in range(6):
            ifire(s, s)
        for s in range(3):
            fire(s, s, s)

        plsc.subcore_barrier()

        cfire(0, 0)
        cfire(1, 1)

        def slot(j, s, fire_g, fire_i, tail_cnt=True):
            drain(s % 3)
            pltpu.sync_copy(rows[s % 3], acc_sh.at[dstb.at[j]], add=True)
            cdrain(s % 2)
            if tail_cnt:
                cfire(j + 2, s % 2)
            if fire_g:
                fire(j + 3, s % 3, (s + 3) % 6)
            if fire_i:
                ifire(j + 6, s % 6)

        @pl.loop(0, 20)
        def _(m):
            j = 6 * m
            for s in range(5):
                slot(j + s, s, True, True)

            slot(j + 5, 5, True, False)

            @pl.when(m < 19)
            def _():
                ifire(j + 11, 5)

        slot(120, 0, True, False)
        slot(121, 1, True, False)
        slot(122, 2, False, False)
        slot(123, 3, False, False, tail_cnt=False)
        slot(124, 4, False, False, tail_cnt=False)

        plsc.subcore_barrier()

        @pl.loop(0, nch)
        def _(j):
            rr = r0 + j * ZCH
            pltpu.sync_copy(acc_sh.at[pl.ds(rr, ZCH)], rows0.at[pl.ds(0, ZCH)])
            pltpu.sync_copy(rows0.at[pl.ds(0, ZCH)], sums_hbm.at[cid, pl.ds(rr, ZCH)])

        @pl.when(last)
        def _():
            pltpu.sync_copy(cnt_sh.at[pl.ds(r0, ROWS_B)], cstg_v.at[pl.ds(0, ROWS_B)])
            pltpu.sync_copy(cstg_v.at[pl.ds(0, ROWS_B)], cnts_hbm.at[pl.ds(cid * N + r0, ROWS_B)])

        @pl.when(jnp.logical_not(last))
        def _():
            pltpu.sync_copy(cnt_sh.at[pl.ds(r0, ROWS_A)], cstg_v)
            pltpu.sync_copy(cstg_v, cnts_hbm.at[pl.ds(cid * N + r0, ROWS_A)])

    return agg(x, src, dst, z128, z1)


def _mlp_body(x_ref, sums_ref, cnt_ref, wlt_ref, wrt_ref, bl_ref,
              w1t_ref, b1_ref, w2t_ref, b2_ref, o_ref):
    s = sums_ref[0] + sums_ref[1]
    c = cnt_ref[0] + cnt_ref[1]
    mean = s / jnp.maximum(c, 1.0)
    h = jnp.dot(mean, wlt_ref[...], preferred_element_type=jnp.float32)
    h = h + jnp.dot(x_ref[...], wrt_ref[...], preferred_element_type=jnp.float32)
    h = h + bl_ref[...]
    h = jnp.where(h >= 0.0, h, 0.01 * h)
    h2 = jnp.dot(h, w1t_ref[...], preferred_element_type=jnp.float32) + b1_ref[...]
    h2 = jnp.maximum(h2, 0.0)
    o_ref[...] = jnp.dot(h2, w2t_ref[...], preferred_element_type=jnp.float32) + b2_ref[...]


def _tc_mlp(x, sums, cnts, wlt, wrt, bl2, w1t, b12, w2t, b22):
    R = 1000
    grid = (N // R,)
    return pl.pallas_call(
        _mlp_body,
        grid=grid,
        in_specs=[
            pl.BlockSpec((R, DIN), lambda i: (i, 0)),
            pl.BlockSpec((NC, R, DIN), lambda i: (0, i, 0)),
            pl.BlockSpec((NC, R, 1), lambda i: (0, i, 0)),
            pl.BlockSpec((DIN, DIN), lambda i: (0, 0)),
            pl.BlockSpec((DIN, DIN), lambda i: (0, 0)),
            pl.BlockSpec((1, DIN), lambda i: (0, 0)),
            pl.BlockSpec((DIN, HID), lambda i: (0, 0)),
            pl.BlockSpec((1, HID), lambda i: (0, 0)),
            pl.BlockSpec((HID, OUT), lambda i: (0, 0)),
            pl.BlockSpec((1, OUT), lambda i: (0, 0)),
        ],
        out_specs=pl.BlockSpec((R, OUT), lambda i: (i, 0)),
        out_shape=jax.ShapeDtypeStruct((N, OUT), jnp.float32),
    )(x, sums, cnts, wlt, wrt, bl2, w1t, b12, w2t, b22)


def kernel(features, edges, edges2, edge_features, additional_feature,
           Wl, bl, Wr, W1, b1, gamma, beta, running_mean, running_var, W2, b2):
    src = edges2[0]
    dst = edges2[1].reshape(NW, NCH, CH)
    z128 = jnp.zeros((ZCH, DIN), jnp.float32)
    z1 = jnp.zeros((ROWS_A,), jnp.float32)
    sums, cnts = _sc_aggregate(features, src, dst, z128, z1)

    scale = gamma / jnp.sqrt(running_var + EPS)
    shift = beta - running_mean * scale
    w2t = (W2 * scale[None, :]).T
    b22 = (b2 + W2 @ shift)[None, :]

    return _tc_mlp(features, sums, cnts.reshape(NC, N, 1), Wl.T, Wr.T,
                   bl[None, :], W1.T, b1[None, :], w2t, b22)


# speedup vs baseline: 1.0204x; 1.0204x over previous
import functools

import jax
import jax.numpy as jnp
from jax import lax
from jax.experimental import pallas as pl
from jax.experimental.pallas import tpu as pltpu
from jax.experimental.pallas import tpu_sc as plsc

N = 10000
E = 320000
DIN = 128
HID = 32
OUT = 3
EPS = 1e-5

NC = 2
NS = 16
NW = NC * NS
E_PER_TILE = E // NW
CH = 80
NCH = E_PER_TILE // CH
ZCH = 80
ROWS_A = 640
ROWS_B = N - (NS - 1) * ROWS_A
NCH_A = ROWS_A // ZCH
NCH_B = ROWS_B // ZCH


def _sc_aggregate(x, src, dst, z128, z1):
    mesh = plsc.VectorSubcoreMesh(core_axis_name="c", subcore_axis_name="s")

    @functools.partial(
        pl.kernel,
        out_type=[
            jax.ShapeDtypeStruct((NC, N, DIN), jnp.float32),
            jax.ShapeDtypeStruct((NC * N,), jnp.float32),
        ],
        mesh=mesh,
        scratch_types=[
            pltpu.VMEM((6, CH), jnp.int32),
            pltpu.VMEM((NCH, CH), jnp.int32),
            pltpu.VMEM((CH, DIN), jnp.float32),
            pltpu.VMEM((CH, DIN), jnp.float32),
            pltpu.VMEM((CH, DIN), jnp.float32),
            pltpu.VMEM((112,), jnp.float32),
            pltpu.VMEM((ROWS_A,), jnp.float32),
            pltpu.VMEM_SHARED((N, DIN), jnp.float32),
            pltpu.VMEM_SHARED((N,), jnp.float32),
            pltpu.SemaphoreType.DMA,
            pltpu.SemaphoreType.DMA,
            pltpu.SemaphoreType.DMA,
            pltpu.SemaphoreType.DMA,
            pltpu.SemaphoreType.DMA,
            pltpu.SemaphoreType.DMA,
            pltpu.SemaphoreType.DMA,
            pltpu.SemaphoreType.DMA,
            pltpu.SemaphoreType.DMA,
            pltpu.SemaphoreType.DMA,
            pltpu.SemaphoreType.DMA,
        ],
    )
    def agg(x_hbm, src_hbm, dst_hbm, z128_hbm, z1_hbm, sums_hbm, cnts_hbm,
            sidx, dstb, rows0, rows1, rows2, ones_v, cstg_v, acc_sh, cnt_sh,
            gs0, gs1, gs2, is0, is1, is2, is3, is4, is5, cs0, cs1):
        cid = lax.axis_index("c")
        sid = lax.axis_index("s")
        wid = cid * NS + sid
        r0 = sid * ROWS_A
        last = sid == NS - 1
        nch = jnp.where(last, NCH_B, NCH_A)

        gsems = [gs0, gs1, gs2]
        isems = [is0, is1, is2, is3, is4, is5]
        csems = [cs0, cs1]
        rows = [rows0, rows1, rows2]

        pltpu.async_copy(dst_hbm.at[wid], dstb, cs0)

        pltpu.sync_copy(z128_hbm, rows0.at[pl.ds(0, ZCH)])
        pltpu.sync_copy(z1_hbm, cstg_v)

        @pl.loop(0, nch)
        def _(j):
            pltpu.async_copy(rows0.at[pl.ds(0, ZCH)],
                             acc_sh.at[pl.ds(r0 + j * ZCH, ZCH)], cs1)

        @pl.when(last)
        def _():
            pltpu.async_copy(cstg_v.at[pl.ds(0, ROWS_B)],
                             cnt_sh.at[pl.ds(r0, ROWS_B)], gs0)

        @pl.when(jnp.logical_not(last))
        def _():
            pltpu.async_copy(cstg_v, cnt_sh.at[pl.ds(r0, ROWS_A)], gs0)

        ovec = jnp.ones((16,), jnp.float32)

        @pl.loop(0, 7)
        def _(i):
            ones_v[pl.ds(i * 16, 16)] = ovec

        @pl.loop(0, nch)
        def _(j):
            pltpu.make_async_copy(rows0.at[pl.ds(0, ZCH)],
                                  acc_sh.at[pl.ds(r0, ZCH)], cs1).wait()

        @pl.when(last)
        def _():
            pltpu.make_async_copy(cstg_v.at[pl.ds(0, ROWS_B)],
                                  cnt_sh.at[pl.ds(r0, ROWS_B)], gs0).wait()

        @pl.when(jnp.logical_not(last))
        def _():
            pltpu.make_async_copy(cstg_v, cnt_sh.at[pl.ds(r0, ROWS_A)], gs0).wait()

        pltpu.make_async_copy(dst_hbm.at[wid], dstb, cs0).wait()

        def ifire(j, b6):
            pltpu.async_copy(src_hbm.at[pl.ds(wid * E_PER_TILE + j * CH, CH)],
                             sidx.at[b6], isems[b6])

        def idrain(b6):
            pltpu.make_async_copy(src_hbm.at[pl.ds(0, CH)],
                                  sidx.at[b6], isems[b6]).wait()

        def fire(j, b3, b6):
            idrain(b6)
            pltpu.async_copy(x_hbm.at[sidx.at[b6]], rows[b3], gsems[b3])

        def drain(b3):
            pltpu.make_async_copy(x_hbm.at[sidx.at[0]], rows[b3],
                                  gsems[b3]).wait()

        def cfire(j, b2):
            pltpu.async_copy(ones_v.at[pl.ds(0, CH)], cnt_sh.at[dstb.at[j]],
                             csems[b2], add=True)

        def cdrain(b2):
            pltpu.make_async_copy(z1_hbm.at[pl.ds(0, CH)],
                                  cstg_v.at[pl.ds(0, CH)], csems[b2]).wait()

        for s in range(6):
            ifire(s, s)
        for s in range(3):
            fire(s, s, s)

        plsc.subcore_barrier()

        cfire(0, 0)
        cfire(1, 1)

        def slot(j, s, fire_g, fire_i, tail_cnt=True):
            drain(s % 3)
            pltpu.sync_copy(rows[s % 3], acc_sh.at[dstb.at[j]], add=True)
            cdrain(s % 2)
            if tail_cnt:
                cfire(j + 2, s % 2)
            if fire_g:
                fire(j + 3, s % 3, (s + 3) % 6)
            if fire_i:
                ifire(j + 6, s % 6)

        @pl.loop(0, 20)
        def _(m):
            j = 6 * m
            for s in range(5):
                slot(j + s, s, True, True)

            slot(j + 5, 5, True, False)

            @pl.when(m < 19)
            def _():
                ifire(j + 11, 5)

        slot(120, 0, True, False)
        slot(121, 1, True, False)
        slot(122, 2, False, False)
        slot(123, 3, False, False, tail_cnt=False)
        slot(124, 4, False, False, tail_cnt=False)

        plsc.subcore_barrier()

        @pl.when(last)
        def _():
            pltpu.async_copy(cnt_sh.at[pl.ds(r0, ROWS_B)],
                             cstg_v.at[pl.ds(0, ROWS_B)], cs0)

        @pl.when(jnp.logical_not(last))
        def _():
            pltpu.async_copy(cnt_sh.at[pl.ds(r0, ROWS_A)], cstg_v, cs0)

        for j in range(NCH_A):
            wbuf = rows[j % 2]
            wsem = gsems[j % 2]
            rr = r0 + j * ZCH

            @pl.when(j < nch)
            def _():
                if j >= 2:
                    pltpu.make_async_copy(wbuf.at[pl.ds(0, ZCH)],
                                          sums_hbm.at[cid, pl.ds(r0, ZCH)],
                                          wsem).wait()
                pltpu.sync_copy(acc_sh.at[pl.ds(rr, ZCH)], wbuf.at[pl.ds(0, ZCH)])
                pltpu.async_copy(wbuf.at[pl.ds(0, ZCH)],
                                 sums_hbm.at[cid, pl.ds(rr, ZCH)], wsem)

        pltpu.make_async_copy(rows0.at[pl.ds(0, ZCH)],
                              sums_hbm.at[cid, pl.ds(r0, ZCH)], gs0).wait()
        pltpu.make_async_copy(rows1.at[pl.ds(0, ZCH)],
                              sums_hbm.at[cid, pl.ds(r0, ZCH)], gs1).wait()

        @pl.when(last)
        def _():
            pltpu.make_async_copy(cnt_sh.at[pl.ds(r0, ROWS_B)],
                                  cstg_v.at[pl.ds(0, ROWS_B)], cs0).wait()
            pltpu.sync_copy(cstg_v.at[pl.ds(0, ROWS_B)], cnts_hbm.at[pl.ds(cid * N + r0, ROWS_B)])

        @pl.when(jnp.logical_not(last))
        def _():
            pltpu.make_async_copy(cnt_sh.at[pl.ds(r0, ROWS_A)], cstg_v, cs0).wait()
            pltpu.sync_copy(cstg_v, cnts_hbm.at[pl.ds(cid * N + r0, ROWS_A)])

    return agg(x, src, dst, z128, z1)


def _mlp_body(x_ref, sums_ref, cnt_ref, wlt_ref, wrt_ref, bl_ref,
              w1t_ref, b1_ref, w2t_ref, b2_ref, o_ref):
    s = sums_ref[0] + sums_ref[1]
    c = cnt_ref[0] + cnt_ref[1]
    mean = s / jnp.maximum(c, 1.0)
    h = jnp.dot(mean, wlt_ref[...], preferred_element_type=jnp.float32)
    h = h + jnp.dot(x_ref[...], wrt_ref[...], preferred_element_type=jnp.float32)
    h = h + bl_ref[...]
    h = jnp.where(h >= 0.0, h, 0.01 * h)
    h2 = jnp.dot(h, w1t_ref[...], preferred_element_type=jnp.float32) + b1_ref[...]
    h2 = jnp.maximum(h2, 0.0)
    o_ref[...] = jnp.dot(h2, w2t_ref[...], preferred_element_type=jnp.float32) + b2_ref[...]


def _tc_mlp(x, sums, cnts, wlt, wrt, bl2, w1t, b12, w2t, b22):
    R = 1000
    grid = (N // R,)
    return pl.pallas_call(
        _mlp_body,
        grid=grid,
        in_specs=[
            pl.BlockSpec((R, DIN), lambda i: (i, 0)),
            pl.BlockSpec((NC, R, DIN), lambda i: (0, i, 0)),
            pl.BlockSpec((NC, R, 1), lambda i: (0, i, 0)),
            pl.BlockSpec((DIN, DIN), lambda i: (0, 0)),
            pl.BlockSpec((DIN, DIN), lambda i: (0, 0)),
            pl.BlockSpec((1, DIN), lambda i: (0, 0)),
            pl.BlockSpec((DIN, HID), lambda i: (0, 0)),
            pl.BlockSpec((1, HID), lambda i: (0, 0)),
            pl.BlockSpec((HID, OUT), lambda i: (0, 0)),
            pl.BlockSpec((1, OUT), lambda i: (0, 0)),
        ],
        out_specs=pl.BlockSpec((R, OUT), lambda i: (i, 0)),
        out_shape=jax.ShapeDtypeStruct((N, OUT), jnp.float32),
    )(x, sums, cnts, wlt, wrt, bl2, w1t, b12, w2t, b22)


def kernel(features, edges, edges2, edge_features, additional_feature,
           Wl, bl, Wr, W1, b1, gamma, beta, running_mean, running_var, W2, b2):
    src = edges2[0]
    dst = edges2[1].reshape(NW, NCH, CH)
    z128 = jnp.zeros((ZCH, DIN), jnp.float32)
    z1 = jnp.zeros((ROWS_A,), jnp.float32)
    sums, cnts = _sc_aggregate(features, src, dst, z128, z1)

    scale = gamma / jnp.sqrt(running_var + EPS)
    shift = beta - running_mean * scale
    w2t = (W2 * scale[None, :]).T
    b22 = (b2 + W2 @ shift)[None, :]

    return _tc_mlp(features, sums, cnts.reshape(NC, N, 1), Wl.T, Wr.T,
                   bl[None, :], W1.T, b1[None, :], w2t, b22)
